# baseline (speedup 1.0000x reference)
import functools

import jax
import jax.numpy as jnp
from jax import lax
from jax.experimental import pallas as pl
from jax.experimental.pallas import tpu as pltpu
from jax.experimental.pallas import tpu_sc as plsc

_B = 16384
_D = 128
_NC = 2
_NS = 16
_NW = _NC * _NS
_RPW = _B // _NW
_CHUNK = 64
_NCHUNK = _RPW // _CHUNK
_LANES = 16
_VPR = _D // _LANES

_mesh = plsc.VectorSubcoreMesh(core_axis_name="c", subcore_axis_name="s")


def _tree_sum(vals):
    vals = list(vals)
    while len(vals) > 1:
        nxt = [vals[i] + vals[i + 1] for i in range(0, len(vals) - 1, 2)]
        if len(vals) % 2:
            nxt.append(vals[-1])
        vals = nxt
    return vals[0]


@functools.partial(
    pl.kernel,
    mesh=_mesh,
    compiler_params=pltpu.CompilerParams(needs_layout_passes=False,
                                         use_tc_tiling_on_sc=False),
    out_type=jax.ShapeDtypeStruct((_B,), jnp.float32),
    scratch_types=[
        pltpu.VMEM((_RPW,), jnp.int32),
        pltpu.VMEM((_RPW,), jnp.int32),
        pltpu.VMEM((2, _CHUNK, _D), jnp.float32),
        pltpu.VMEM((2, _CHUNK, _D), jnp.float32),
        pltpu.VMEM((_RPW,), jnp.float32),
        pltpu.VMEM((_CHUNK * _LANES,), jnp.float32),
        pltpu.SemaphoreType.DMA((2,)),
        pltpu.SemaphoreType.DMA((2,)),
    ],
)
def _sc_dot(uid_hbm, pid_hbm, ut_hbm, pt_hbm, out_hbm,
            uidx_v, pidx_v, urows_v, prows_v, out_v, tile_v, sem_u, sem_p):
    wid = lax.axis_index("s") * _NC + lax.axis_index("c")
    base = wid * _RPW
    pltpu.sync_copy(uid_hbm.at[pl.ds(base, _RPW)], uidx_v)
    pltpu.sync_copy(pid_hbm.at[pl.ds(base, _RPW)], pidx_v)
    lane_ids = lax.iota(jnp.int32, _LANES)
    cols = [lane_ids * _LANES + l for l in range(_LANES)]

    def descriptors(ci, b):
        idx_u = uidx_v.at[pl.ds(ci * _CHUNK, _CHUNK)]
        idx_p = pidx_v.at[pl.ds(ci * _CHUNK, _CHUNK)]
        cu = pltpu.make_async_copy(ut_hbm.at[idx_u], urows_v.at[b],
                                   sem_u.at[b])
        cp = pltpu.make_async_copy(pt_hbm.at[idx_p], prows_v.at[b],
                                   sem_p.at[b])
        return (cu, cp)

    def issue(ci, b):
        cu, cp = descriptors(ci, b)
        cu.start()
        cp.start()

    issue(0, 0)

    def chunk_body(ci, _):
        b = lax.rem(ci, 2)

        @pl.when(ci + 1 < _NCHUNK)
        def _():
            issue(ci + 1, 1 - b)

        cu, cp = descriptors(ci, b)
        cu.wait()
        cp.wait()

        @plsc.parallel_loop(0, _CHUNK, 1, unroll=8)
        def _row_body(r):
            prods = [(urows_v[b, r, pl.ds(j * _LANES, _LANES)]
                      * prows_v[b, r, pl.ds(j * _LANES, _LANES)])
                     for j in range(_VPR)]
            tile_v[pl.ds(r * _LANES, _LANES)] = _tree_sum(prods)

        @plsc.parallel_loop(0, _CHUNK // _LANES, 1, unroll=2)
        def _red_body(g):
            gb = g * (_LANES * _LANES)
            tot = _tree_sum([plsc.load_gather(tile_v, [gb + cols[l]])
                             for l in range(_LANES)])
            out_v[pl.ds(ci * _CHUNK + g * _LANES, _LANES)] = tot

        return 0

    lax.fori_loop(0, _NCHUNK, chunk_body, 0)
    pltpu.sync_copy(out_v, out_hbm.at[pl.ds(base, _RPW)])


def kernel(user_ids, place_ids, user_table, place_table):
    out = _sc_dot(user_ids.astype(jnp.int32), place_ids.astype(jnp.int32),
                  user_table, place_table)
    return out.reshape(_B, 1)


# speedup vs baseline: 1.0021x; 1.0021x over previous
import functools

import jax
import jax.numpy as jnp
from jax import lax
from jax.experimental import pallas as pl
from jax.experimental.pallas import tpu as pltpu
from jax.experimental.pallas import tpu_sc as plsc

_B = 16384
_D = 128
_NC = 2
_NS = 16
_NW = _NC * _NS
_RPW = _B // _NW
_CHUNK = 64
_NCHUNK = _RPW // _CHUNK
_LANES = 16
_VPR = _D // _LANES

_mesh = plsc.VectorSubcoreMesh(core_axis_name="c", subcore_axis_name="s")


def _tree_sum(vals):
    vals = list(vals)
    while len(vals) > 1:
        nxt = [vals[i] + vals[i + 1] for i in range(0, len(vals) - 1, 2)]
        if len(vals) % 2:
            nxt.append(vals[-1])
        vals = nxt
    return vals[0]


@functools.partial(
    pl.kernel,
    mesh=_mesh,
    compiler_params=pltpu.CompilerParams(needs_layout_passes=False),
    out_type=jax.ShapeDtypeStruct((_B,), jnp.float32),
    scratch_types=[
        pltpu.VMEM((_RPW,), jnp.int32),
        pltpu.VMEM((_RPW,), jnp.int32),
        pltpu.VMEM((2, _CHUNK, _D), jnp.float32),
        pltpu.VMEM((2, _CHUNK, _D), jnp.float32),
        pltpu.VMEM((_RPW,), jnp.float32),
        pltpu.VMEM((_CHUNK * _LANES,), jnp.float32),
        pltpu.SemaphoreType.DMA((2,)),
        pltpu.SemaphoreType.DMA((2,)),
    ],
)
def _sc_dot(uid_hbm, pid_hbm, ut_hbm, pt_hbm, out_hbm,
            uidx_v, pidx_v, urows_v, prows_v, out_v, tile_v, sem_u, sem_p):
    wid = lax.axis_index("s") * _NC + lax.axis_index("c")
    base = wid * _RPW
    pltpu.sync_copy(uid_hbm.at[pl.ds(base, _RPW)], uidx_v)
    pltpu.sync_copy(pid_hbm.at[pl.ds(base, _RPW)], pidx_v)
    lane_ids = lax.iota(jnp.int32, _LANES)
    cols = [lane_ids * _LANES + l for l in range(_LANES)]

    def descriptors(ci, b):
        idx_u = uidx_v.at[pl.ds(ci * _CHUNK, _CHUNK)]
        idx_p = pidx_v.at[pl.ds(ci * _CHUNK, _CHUNK)]
        cu = pltpu.make_async_copy(ut_hbm.at[idx_u], urows_v.at[b],
                                   sem_u.at[b])
        cp = pltpu.make_async_copy(pt_hbm.at[idx_p], prows_v.at[b],
                                   sem_p.at[b])
        return (cu, cp)

    def issue(ci, b):
        cu, cp = descriptors(ci, b)
        cu.start()
        cp.start()

    issue(0, 0)

    def chunk_body(ci, _):
        b = lax.rem(ci, 2)

        @pl.when(ci + 1 < _NCHUNK)
        def _():
            issue(ci + 1, 1 - b)

        cu, cp = descriptors(ci, b)
        cu.wait()
        cp.wait()

        @plsc.parallel_loop(0, _CHUNK, 1, unroll=8)
        def _row_body(r):
            prods = [(urows_v[b, r, pl.ds(j * _LANES, _LANES)]
                      * prows_v[b, r, pl.ds(j * _LANES, _LANES)])
                     for j in range(_VPR)]
            tile_v[pl.ds(r * _LANES, _LANES)] = _tree_sum(prods)

        @plsc.parallel_loop(0, _CHUNK // _LANES, 1, unroll=2)
        def _red_body(g):
            gb = g * (_LANES * _LANES)
            tot = _tree_sum([plsc.load_gather(tile_v, [gb + cols[l]])
                             for l in range(_LANES)])
            out_v[pl.ds(ci * _CHUNK + g * _LANES, _LANES)] = tot

        return 0

    lax.fori_loop(0, _NCHUNK, chunk_body, 0)
    pltpu.sync_copy(out_v, out_hbm.at[pl.ds(base, _RPW)])


def kernel(user_ids, place_ids, user_table, place_table):
    out = _sc_dot(user_ids.astype(jnp.int32), place_ids.astype(jnp.int32),
                  user_table, place_table)
    return out.reshape(_B, 1)


# speedup vs baseline: 1.0059x; 1.0038x over previous
import functools

import jax
import jax.numpy as jnp
from jax import lax
from jax.experimental import pallas as pl
from jax.experimental.pallas import tpu as pltpu
from jax.experimental.pallas import tpu_sc as plsc

_B = 16384
_D = 128
_NC = 2
_NS = 16
_NW = _NC * _NS
_RPW = _B // _NW
_CHUNK = 64
_NCHUNK = _RPW // _CHUNK
_LANES = 16
_VPR = _D // _LANES

_mesh = plsc.VectorSubcoreMesh(core_axis_name="c", subcore_axis_name="s")


def _tree_sum(vals):
    vals = list(vals)
    while len(vals) > 1:
        nxt = [vals[i] + vals[i + 1] for i in range(0, len(vals) - 1, 2)]
        if len(vals) % 2:
            nxt.append(vals[-1])
        vals = nxt
    return vals[0]


@functools.partial(
    pl.kernel,
    mesh=_mesh,
    compiler_params=pltpu.CompilerParams(needs_layout_passes=False),
    out_type=jax.ShapeDtypeStruct((_B,), jnp.float32),
    scratch_types=[
        pltpu.VMEM((_RPW,), jnp.int32),
        pltpu.VMEM((_RPW,), jnp.int32),
        pltpu.VMEM((2, _CHUNK, _D), jnp.float32),
        pltpu.VMEM((2, _CHUNK, _D), jnp.float32),
        pltpu.VMEM((_RPW,), jnp.float32),
        pltpu.VMEM((_CHUNK * _LANES,), jnp.float32),
        pltpu.SemaphoreType.DMA((2,)),
        pltpu.SemaphoreType.DMA((2,)),
    ],
)
def _sc_dot(uid_hbm, pid_hbm, ut_hbm, pt_hbm, out_hbm,
            uidx_v, pidx_v, urows_v, prows_v, out_v, tile_v, sem_u, sem_p):
    wid = lax.axis_index("s") * _NC + lax.axis_index("c")
    base = wid * _RPW
    pltpu.sync_copy(uid_hbm.at[pl.ds(base, _RPW)], uidx_v)
    pltpu.sync_copy(pid_hbm.at[pl.ds(base, _RPW)], pidx_v)
    lane_ids = lax.iota(jnp.int32, _LANES)
    cols = [lane_ids * _LANES + l for l in range(_LANES)]

    def descriptors(ci, b):
        idx_u = uidx_v.at[pl.ds(ci * _CHUNK, _CHUNK)]
        idx_p = pidx_v.at[pl.ds(ci * _CHUNK, _CHUNK)]
        cu = pltpu.make_async_copy(ut_hbm.at[idx_u], urows_v.at[b],
                                   sem_u.at[b])
        cp = pltpu.make_async_copy(pt_hbm.at[idx_p], prows_v.at[b],
                                   sem_p.at[b])
        return (cu, cp)

    def issue(ci, b):
        cu, cp = descriptors(ci, b)
        cu.start()
        cp.start()

    issue(0, 0)

    def chunk_body(ci, _):
        b = lax.rem(ci, 2)

        @pl.when(ci + 1 < _NCHUNK)
        def _():
            issue(ci + 1, 1 - b)

        cu, cp = descriptors(ci, b)
        cu.wait()
        cp.wait()

        @plsc.parallel_loop(0, _CHUNK, 1, unroll=8)
        def _row_body(r):
            prods = [(urows_v[b, r, pl.ds(j * _LANES, _LANES)]
                      * prows_v[b, r, pl.ds(j * _LANES, _LANES)])
                     for j in range(_VPR)]
            tile_v[pl.ds(r * _LANES, _LANES)] = _tree_sum(prods)

        @plsc.parallel_loop(0, _CHUNK // _LANES, 1, unroll=4)
        def _red_body(g):
            gb = g * (_LANES * _LANES)
            tot = _tree_sum([plsc.load_gather(tile_v, [gb + cols[l]])
                             for l in range(_LANES)])
            out_v[pl.ds(ci * _CHUNK + g * _LANES, _LANES)] = tot

        return 0

    lax.fori_loop(0, _NCHUNK, chunk_body, 0)
    pltpu.sync_copy(out_v, out_hbm.at[pl.ds(base, _RPW)])


def kernel(user_ids, place_ids, user_table, place_table):
    out = _sc_dot(user_ids.astype(jnp.int32), place_ids.astype(jnp.int32),
                  user_table, place_table)
    return out.reshape(_B, 1)
